# baseline (speedup 1.0000x reference)
import functools

import jax
import jax.numpy as jnp
from jax import lax
from jax.experimental import pallas as pl
from jax.experimental.pallas import tpu as pltpu
from jax.experimental.pallas import tpu_sc as plsc

H = 128
EPS = 1e-12
GATHER_W = 128
B_BLK = 8


def _sc_gather(word_table, ids_2d, n):
    @functools.partial(
        pl.kernel,
        out_type=jax.ShapeDtypeStruct((n, word_table.shape[1]), word_table.dtype),
        mesh=plsc.VectorSubcoreMesh(core_axis_name="core",
                                    subcore_axis_name="subcore"),
    )
    def k(table_hbm, i_hbm, o_hbm):
        def body(i_vmem, o_vmem):
            pltpu.sync_copy(table_hbm.at[i_vmem.at[0]], o_vmem)

        pltpu.emit_pipeline(
            body,
            grid=(n // GATHER_W,),
            in_specs=[pl.BlockSpec((1, GATHER_W), index_map=lambda i: (0, i))],
            out_specs=[pl.BlockSpec((GATHER_W, word_table.shape[1]),
                                    index_map=lambda i: (i, 0))],
            core_axis_name=("core", "subcore"),
            dimension_semantics=(pltpu.PARALLEL,),
        )(i_hbm, o_hbm)

    return k(word_table, ids_2d)


def _ln_body(g_ref, pos_ref, tok_ref, gamma_ref, beta_ref, o_ref):
    x = (g_ref[...]
         + pos_ref[...][None, :, :] + tok_ref[...][0][None, None, :])
    nb, s, _ = x.shape
    x2d = x.reshape(nb * s, H)
    ones = jnp.ones((H, H), dtype=jnp.bfloat16)
    xb = x2d.astype(jnp.bfloat16)
    s1 = jax.lax.dot(xb, ones, precision=lax.Precision.DEFAULT,
                     preferred_element_type=jnp.float32)
    s2 = jax.lax.dot(xb * xb, ones, precision=lax.Precision.DEFAULT,
                     preferred_element_type=jnp.float32)
    mean = s1 * (1.0 / H)
    var = s2 * (1.0 / H) - mean * mean
    inv = lax.rsqrt(var + EPS)
    a = inv * gamma_ref[...][0][None, :]
    b = beta_ref[...][0][None, :] - mean * a
    o_ref[...] = (x2d * a + b).reshape(nb, s, H)


def _tc_ln(g, pos, tok, gamma2d, beta2d):
    B, S, _ = g.shape
    return pl.pallas_call(
        _ln_body,
        grid=(B // B_BLK,),
        in_specs=[
            pl.BlockSpec((B_BLK, S, H), lambda i: (i, 0, 0)),
            pl.BlockSpec((S, H), lambda i: (0, 0)),
            pl.BlockSpec((2, H), lambda i: (0, 0)),
            pl.BlockSpec((1, H), lambda i: (0, 0)),
            pl.BlockSpec((1, H), lambda i: (0, 0)),
        ],
        out_specs=pl.BlockSpec((B_BLK, S, H), lambda i: (i, 0, 0)),
        out_shape=jax.ShapeDtypeStruct((B, S, H), jnp.float32),
    )(g, pos, tok, gamma2d, beta2d)


N_CHUNKS = 8


def kernel(input_ids, word_table, pos_table, tok_table, gamma, beta):
    B, S = input_ids.shape
    ids = input_ids.astype(jnp.int32)
    gamma2d, beta2d = gamma.reshape(1, H), beta.reshape(1, H)
    bc = B // N_CHUNKS
    outs = []
    for k in range(N_CHUNKS):
        ids_k = ids[k * bc:(k + 1) * bc].reshape(1, bc * S)
        g = _sc_gather(word_table, ids_k, bc * S).reshape(bc, S, H)
        outs.append(_tc_ln(g, pos_table, tok_table, gamma2d, beta2d))
    return jnp.concatenate(outs, axis=0)


# speedup vs baseline: 1.3972x; 1.3972x over previous
import functools

import jax
import jax.numpy as jnp
from jax import lax
from jax.experimental import pallas as pl
from jax.experimental.pallas import tpu as pltpu
from jax.experimental.pallas import tpu_sc as plsc

H = 128
EPS = 1e-12
GATHER_W = 128
B_BLK = 8


def _sc_gather(word_table, ids_2d, n):
    @functools.partial(
        pl.kernel,
        out_type=jax.ShapeDtypeStruct((n, word_table.shape[1]), word_table.dtype),
        mesh=plsc.VectorSubcoreMesh(core_axis_name="core",
                                    subcore_axis_name="subcore"),
    )
    def k(table_hbm, i_hbm, o_hbm):
        def body(i_vmem, o_vmem):
            pltpu.sync_copy(table_hbm.at[i_vmem.at[0]], o_vmem)

        pltpu.emit_pipeline(
            body,
            grid=(n // GATHER_W,),
            in_specs=[pl.BlockSpec((1, GATHER_W), index_map=lambda i: (0, i))],
            out_specs=[pl.BlockSpec((GATHER_W, word_table.shape[1]),
                                    index_map=lambda i: (i, 0))],
            core_axis_name=("core", "subcore"),
            dimension_semantics=(pltpu.PARALLEL,),
        )(i_hbm, o_hbm)

    return k(word_table, ids_2d)


def _ln_body(g_ref, pos_ref, tok_ref, gamma_ref, beta_ref, o_ref):
    x = (g_ref[...]
         + pos_ref[...][None, :, :] + tok_ref[...][0][None, None, :])
    nb, s, _ = x.shape
    x2d = x.reshape(nb * s, H)
    ones = jnp.ones((H, H), dtype=jnp.bfloat16)
    xb = x2d.astype(jnp.bfloat16)
    s1 = jax.lax.dot(xb, ones, precision=lax.Precision.DEFAULT,
                     preferred_element_type=jnp.float32)
    s2 = jax.lax.dot(xb * xb, ones, precision=lax.Precision.DEFAULT,
                     preferred_element_type=jnp.float32)
    mean = s1 * (1.0 / H)
    var = s2 * (1.0 / H) - mean * mean
    inv = lax.rsqrt(var + EPS)
    a = inv * gamma_ref[...][0][None, :]
    b = beta_ref[...][0][None, :] - mean * a
    o_ref[...] = (x2d * a + b).reshape(nb, s, H)


def _ln_body_acc(g_ref, pos_ref, tok_ref, gamma_ref, beta_ref, acc_ref,
                 o_ref):
    del acc_ref
    _ln_body(g_ref, pos_ref, tok_ref, gamma_ref, beta_ref, o_ref)


def _tc_ln_chunk(g, pos, tok, gamma2d, beta2d, out_b, chunk, buf):
    bc, S, _ = g.shape
    base = chunk * (bc // B_BLK)
    common = [
        pl.BlockSpec((S, H), lambda i: (0, 0)),
        pl.BlockSpec((2, H), lambda i: (0, 0)),
        pl.BlockSpec((1, H), lambda i: (0, 0)),
        pl.BlockSpec((1, H), lambda i: (0, 0)),
    ]
    in_specs = [pl.BlockSpec((B_BLK, S, H), lambda i: (i, 0, 0))] + common
    args = [g, pos, tok, gamma2d, beta2d]
    body = _ln_body
    kwargs = {}
    if buf is not None:
        in_specs.append(pl.BlockSpec(memory_space=pl.ANY))
        args.append(buf)
        body = _ln_body_acc
        kwargs = dict(input_output_aliases={5: 0})
    return pl.pallas_call(
        body,
        grid=(bc // B_BLK,),
        in_specs=in_specs,
        out_specs=pl.BlockSpec((B_BLK, S, H), lambda i: (base + i, 0, 0)),
        out_shape=jax.ShapeDtypeStruct((out_b, S, H), jnp.float32),
        **kwargs,
    )(*args)


N_CHUNKS = 4


def kernel(input_ids, word_table, pos_table, tok_table, gamma, beta):
    B, S = input_ids.shape
    ids = input_ids.astype(jnp.int32)
    gamma2d, beta2d = gamma.reshape(1, H), beta.reshape(1, H)
    bc = B // N_CHUNKS
    gs = []
    for k in range(N_CHUNKS):
        ids_k = ids[k * bc:(k + 1) * bc].reshape(1, bc * S)
        gs.append(_sc_gather(word_table, ids_k, bc * S).reshape(bc, S, H))
    out = None
    for k in range(N_CHUNKS):
        out = _tc_ln_chunk(gs[k], pos_table, tok_table, gamma2d, beta2d,
                           B, k, out)
    return out
